# baseline (speedup 1.0000x reference)
import jax
import jax.numpy as jnp
import numpy as np
from jax.experimental import pallas as pl
from jax.experimental.pallas import tpu as pltpu

L0, L1, L2 = 64, 32, 16
G, NPG = 100, 100
N = G * NPG
D = L0 + 3 * L1 + 5 * L2
B = 20
PAD = 128

_PREC = jax.lax.Precision.DEFAULT


def _dot(a, b):
    return jnp.dot(a, b, precision=_PREC, preferred_element_type=jnp.float32)


def _block_kernel(x_ref, pos_ref, posT_ref,
                  A_ref, W0_ref, P1_ref, P2_ref,
                  T1_ref, T2_ref, Wg1_ref, Wg2_ref,
                  Wr_ref, br_ref, wd_ref, o_ref):
    x = x_ref[:]

    Y = _dot(x, A_ref[:])
    p1f = _dot(Y[:, :L0], P1_ref[:])
    p2f = _dot(Y[:, :L0], P2_ref[:])

    WrA = Wr_ref[:]
    brA = br_ref[:]
    s3 = float(np.sqrt(3.0))
    ECOEF = 16.0 / 7.0
    KCH = [float(np.exp(-2.0 * ((4.0 * b / 7.0) ** 2
                                - (4.0 * (b - 1) / 7.0) ** 2)))
           for b in range(1, 8)]

    c0l, c1al, c2al = [], [], []
    c1bl = [[] for _ in range(3)]
    c2bl = [[] for _ in range(5)]
    CHUNKS = [(0, 32), (32, 32), (64, 32), (96, 4)]
    for q in range(B):
        base = q * NPG
        wp = [[] for _ in range(3)]
        t1p = [[] for _ in range(3)]
        t2p = [[] for _ in range(5)]
        for r0, ch in CHUNKS:
            rel = [posT_ref[k:k + 1, q * PAD:q * PAD + NPG]
                   - pos_ref[base + r0:base + r0 + ch, k:k + 1]
                   for k in range(3)]
            d2 = rel[0] * rel[0] + rel[1] * rel[1] + rel[2] * rel[2] + 1e-9
            inv = jax.lax.rsqrt(d2)
            dist = d2 * inv

            bas = jnp.exp(d2 * -2.0)
            E = jnp.exp(dist * ECOEF)
            wch = [brA[0:1, m:m + 1] + bas * WrA[0:1, m:m + 1]
                   for m in range(5)]
            for b in range(1, 8):
                bas = (bas * E) * KCH[b - 1]
                for m in range(5):
                    wch[m] = wch[m] + bas * WrA[b:b + 1, m:m + 1]

            w3i = wch[3] * inv
            w4i2 = wch[4] * (inv * inv)
            s3w = s3 * w4i2
            u = s3w * rel[0]
            sy = s3w * rel[1]
            v = w4i2 * rel[2]
            vz = v * rel[2]
            for m in range(3):
                wp[m].append(wch[m])
            for k in range(3):
                t1p[k].append(w3i * rel[k])
            for m, tt in enumerate(
                    [u * rel[1], sy * rel[2], 1.5 * vz - 0.5 * wch[4],
                     u * rel[2], 0.5 * (u * rel[0] - sy * rel[1])]):
                t2p[m].append(tt)
        wch = [jnp.concatenate(wp[m], axis=0) for m in range(3)]
        t1 = [jnp.concatenate(t1p[k], axis=0) for k in range(3)]
        t2 = [jnp.concatenate(t2p[m], axis=0) for m in range(5)]
        sl = slice(base, base + NPG)

        c0l.append(_dot(wch[0], Y[sl, :L0]))
        c1al.append(_dot(wch[1], Y[sl, L0:L0 + 3 * L1]))
        c2al.append(_dot(wch[2], Y[sl, L0 + 3 * L1:]))
        for k in range(3):
            c1bl[k].append(_dot(t1[k], p1f[sl]))
        for m in range(5):
            c2bl[m].append(_dot(t2[m], p2f[sl]))

    c0 = jnp.concatenate(c0l, axis=0)
    c1a = jnp.concatenate(c1al, axis=0)
    c2a = jnp.concatenate(c2al, axis=0)

    wd = wd_ref[:]
    c0 = c0 - wd[0:1, 0:1] * Y[:, :L0]
    c1a = c1a - wd[0:1, 1:2] * Y[:, L0:L0 + 3 * L1]
    c2a = c2a - wd[0:1, 2:3] * Y[:, L0 + 3 * L1:]
    c2b2 = jnp.concatenate(c2bl[2], axis=0) + (0.5 * wd[0:1, 4:5]) * p2f

    T1 = T1_ref[:]
    T2 = T2_ref[:]
    o1 = _dot(c1a, T1)
    for k in range(3):
        o1 = o1 + _dot(jnp.concatenate(c1bl[k], axis=0),
                       T1[k * L1:(k + 1) * L1])
    o2 = _dot(c2a, T2)
    for m in range(5):
        cm = c2b2 if m == 2 else jnp.concatenate(c2bl[m], axis=0)
        o2 = o2 + _dot(cm, T2[m * L2:(m + 1) * L2])
    m0 = x[:, :L0] + _dot(c0, W0_ref[:])
    g1 = jax.nn.sigmoid(_dot(m0, Wg1_ref[:]))
    g2 = jax.nn.sigmoid(_dot(m0, Wg2_ref[:]))
    o_ref[:] = jnp.concatenate(
        [m0 * jax.nn.sigmoid(m0),
         (x[:, L0:L0 + 3 * L1] + o1) * g1,
         (x[:, L0 + 3 * L1:] + o2) * g2], axis=1)


def kernel(nodes, pos, batch, W0, W1, W2, P1, P2, Wr, br, Wg1, Wg2):
    del batch
    posTg = pos.reshape(G, NPG, 3).transpose(0, 2, 1)
    posT = jnp.pad(posTg, ((0, 0), (0, 0), (0, PAD - NPG)))
    posT = posT.transpose(1, 0, 2).reshape(3, G * PAD)

    s = 1.0 / np.sqrt(NPG)
    Wr2 = Wr * s
    br2 = br.reshape(1, 5) * s
    d0 = np.sqrt(1e-9)
    bas0 = np.exp(-2.0 * (d0 - np.linspace(0.0, 4.0, 8)) ** 2)
    wd = bas0[None, :].astype(np.float32) @ Wr2 + br2

    I3 = jnp.eye(3, dtype=jnp.float32)
    I5 = jnp.eye(5, dtype=jnp.float32)
    A = jnp.zeros((D, D), dtype=jnp.float32)
    A = A.at[:L0, :L0].set(W0)
    A = A.at[L0:L0 + 3 * L1, L0:L0 + 3 * L1].set(
        jnp.einsum('cd,ik->cikd', W1, I3).reshape(3 * L1, 3 * L1))
    A = A.at[L0 + 3 * L1:, L0 + 3 * L1:].set(
        jnp.einsum('cd,ik->cikd', W2, I5).reshape(5 * L2, 5 * L2))
    T1cat = jnp.einsum('cd,kq->kcdq', W1, I3).reshape(3 * L1, 3 * L1)
    T2cat = jnp.einsum('cd,kq->kcdq', W2, I5).reshape(5 * L2, 5 * L2)
    Wg1rep = jnp.einsum('ac,k->ack', Wg1, jnp.ones(3)).reshape(L0, 3 * L1)
    Wg2rep = jnp.einsum('ac,k->ack', Wg2, jnp.ones(5)).reshape(L0, 5 * L2)

    full = lambda shape: pl.BlockSpec(shape, lambda g: (0,) * len(shape))
    out = pl.pallas_call(
        _block_kernel,
        grid=(G // B,),
        in_specs=[
            pl.BlockSpec((B * NPG, D), lambda g: (g, 0)),
            pl.BlockSpec((B * NPG, 3), lambda g: (g, 0)),
            pl.BlockSpec((3, B * PAD), lambda g: (0, g)),
            full((D, D)), full((L0, L0)), full((L0, L1)), full((L0, L2)),
            full((3 * L1, 3 * L1)), full((5 * L2, 5 * L2)),
            full((L0, 3 * L1)), full((L0, 5 * L2)),
            full((8, 5)), full((1, 5)), full((1, 5)),
        ],
        out_specs=pl.BlockSpec((B * NPG, D), lambda g: (g, 0)),
        out_shape=jax.ShapeDtypeStruct((N, D), jnp.float32),
        compiler_params=pltpu.CompilerParams(
            dimension_semantics=("parallel",)),
    )(nodes, pos, posT, A, W0, P1, P2, T1cat, T2cat, Wg1rep, Wg2rep,
      Wr2, br2, wd)

    return out


# speedup vs baseline: 1.0421x; 1.0421x over previous
import jax
import jax.numpy as jnp
import numpy as np
from jax.experimental import pallas as pl
from jax.experimental.pallas import tpu as pltpu

L0, L1, L2 = 64, 32, 16
G, NPG = 100, 100
N = G * NPG
D = L0 + 3 * L1 + 5 * L2
B = 20
PAD = 128

_PREC = jax.lax.Precision.DEFAULT


def _dot(a, b):
    return jnp.dot(a, b, precision=_PREC, preferred_element_type=jnp.float32)


def _block_kernel(x_ref, pos_ref, posT_ref, Wp_ref, o_ref):
    x = x_ref[:]
    Wp = Wp_ref[:]
    A = Wp[0:240, :]
    W0 = Wp[240:304, 0:L0]
    T1 = Wp[304:400, 0:3 * L1]
    T2 = Wp[400:480, 0:5 * L2]
    Wg1 = Wp[480:544, 0:3 * L1]
    Wg2 = Wp[544:608, 0:5 * L2]
    P1 = Wp[608:672, 0:L1]
    P2 = Wp[672:736, 0:L2]

    Y = _dot(x, A)
    p1f = _dot(Y[:, :L0], P1)
    p2f = _dot(Y[:, :L0], P2)

    WrA = Wp[736:744, 0:5]
    brA = Wp[744:745, 0:5]
    s3 = float(np.sqrt(3.0))
    ECOEF = 16.0 / 7.0
    KCH = [float(np.exp(-2.0 * ((4.0 * b / 7.0) ** 2
                                - (4.0 * (b - 1) / 7.0) ** 2)))
           for b in range(1, 8)]

    c0l, c1al, c2al = [], [], []
    c1bl = [[] for _ in range(3)]
    c2bl = [[] for _ in range(5)]
    CHUNKS = [(0, 32), (32, 32), (64, 32), (96, 4)]
    for q in range(B):
        base = q * NPG
        wp = [[] for _ in range(3)]
        t1p = [[] for _ in range(3)]
        t2p = [[] for _ in range(5)]
        for r0, ch in CHUNKS:
            rel = [posT_ref[k:k + 1, q * PAD:q * PAD + NPG]
                   - pos_ref[base + r0:base + r0 + ch, k:k + 1]
                   for k in range(3)]
            d2 = rel[0] * rel[0] + rel[1] * rel[1] + rel[2] * rel[2] + 1e-9
            inv = jax.lax.rsqrt(d2)
            dist = d2 * inv

            bas = jnp.exp(d2 * -2.0)
            E = jnp.exp(dist * ECOEF)
            wch = [brA[0:1, m:m + 1] + bas * WrA[0:1, m:m + 1]
                   for m in range(5)]
            for b in range(1, 8):
                bas = (bas * E) * KCH[b - 1]
                for m in range(5):
                    wch[m] = wch[m] + bas * WrA[b:b + 1, m:m + 1]

            w3i = wch[3] * inv
            w4i2 = wch[4] * (inv * inv)
            s3w = s3 * w4i2
            u = s3w * rel[0]
            sy = s3w * rel[1]
            v = w4i2 * rel[2]
            vz = v * rel[2]
            for m in range(3):
                wp[m].append(wch[m])
            for k in range(3):
                t1p[k].append(w3i * rel[k])
            for m, tt in enumerate(
                    [u * rel[1], sy * rel[2], 1.5 * vz - 0.5 * wch[4],
                     u * rel[2], 0.5 * (u * rel[0] - sy * rel[1])]):
                t2p[m].append(tt)
        wch = [jnp.concatenate(wp[m], axis=0) for m in range(3)]
        t1 = [jnp.concatenate(t1p[k], axis=0) for k in range(3)]
        t2 = [jnp.concatenate(t2p[m], axis=0) for m in range(5)]
        sl = slice(base, base + NPG)

        c0l.append(_dot(wch[0], Y[sl, :L0]))
        c1al.append(_dot(wch[1], Y[sl, L0:L0 + 3 * L1]))
        c2al.append(_dot(wch[2], Y[sl, L0 + 3 * L1:]))
        for k in range(3):
            c1bl[k].append(_dot(t1[k], p1f[sl]))
        for m in range(5):
            c2bl[m].append(_dot(t2[m], p2f[sl]))

    c0 = jnp.concatenate(c0l, axis=0)
    c1a = jnp.concatenate(c1al, axis=0)
    c2a = jnp.concatenate(c2al, axis=0)

    wd = Wp[745:746, 0:5]
    c0 = c0 - wd[0:1, 0:1] * Y[:, :L0]
    c1a = c1a - wd[0:1, 1:2] * Y[:, L0:L0 + 3 * L1]
    c2a = c2a - wd[0:1, 2:3] * Y[:, L0 + 3 * L1:]
    c2b2 = jnp.concatenate(c2bl[2], axis=0) + (0.5 * wd[0:1, 4:5]) * p2f

    o1 = _dot(c1a, T1)
    for k in range(3):
        o1 = o1 + _dot(jnp.concatenate(c1bl[k], axis=0),
                       T1[k * L1:(k + 1) * L1])
    o2 = _dot(c2a, T2)
    for m in range(5):
        cm = c2b2 if m == 2 else jnp.concatenate(c2bl[m], axis=0)
        o2 = o2 + _dot(cm, T2[m * L2:(m + 1) * L2])
    m0 = x[:, :L0] + _dot(c0, W0)
    g1 = jax.nn.sigmoid(_dot(m0, Wg1))
    g2 = jax.nn.sigmoid(_dot(m0, Wg2))
    o_ref[:] = jnp.concatenate(
        [m0 * jax.nn.sigmoid(m0),
         (x[:, L0:L0 + 3 * L1] + o1) * g1,
         (x[:, L0 + 3 * L1:] + o2) * g2], axis=1)


def kernel(nodes, pos, batch, W0, W1, W2, P1, P2, Wr, br, Wg1, Wg2):
    del batch
    posTg = pos.reshape(G, NPG, 3).transpose(0, 2, 1)
    posT = jnp.pad(posTg, ((0, 0), (0, 0), (0, PAD - NPG)))
    posT = posT.transpose(1, 0, 2).reshape(3, G * PAD)

    s = 1.0 / np.sqrt(NPG)
    Wr2 = Wr * s
    br2 = br.reshape(1, 5) * s
    d0 = np.sqrt(1e-9)
    bas0 = np.exp(-2.0 * (d0 - np.linspace(0.0, 4.0, 8)) ** 2)
    wd = bas0[None, :].astype(np.float32) @ Wr2 + br2

    I3 = jnp.eye(3, dtype=jnp.float32)
    I5 = jnp.eye(5, dtype=jnp.float32)
    A = jnp.zeros((D, D), dtype=jnp.float32)
    A = A.at[:L0, :L0].set(W0)
    A = A.at[L0:L0 + 3 * L1, L0:L0 + 3 * L1].set(
        jnp.einsum('cd,ik->cikd', W1, I3).reshape(3 * L1, 3 * L1))
    A = A.at[L0 + 3 * L1:, L0 + 3 * L1:].set(
        jnp.einsum('cd,ik->cikd', W2, I5).reshape(5 * L2, 5 * L2))
    T1cat = jnp.einsum('cd,kq->kcdq', W1, I3).reshape(3 * L1, 3 * L1)
    T2cat = jnp.einsum('cd,kq->kcdq', W2, I5).reshape(5 * L2, 5 * L2)
    Wg1rep = jnp.einsum('ac,k->ack', Wg1, jnp.ones(3)).reshape(L0, 3 * L1)
    Wg2rep = jnp.einsum('ac,k->ack', Wg2, jnp.ones(5)).reshape(L0, 5 * L2)

    cpad = lambda a: jnp.pad(a, ((0, 0), (0, D - a.shape[1])))
    brwd = jnp.pad(jnp.concatenate([br2, wd], axis=0), ((0, 6), (0, D - 5)))
    Wpack = jnp.concatenate(
        [A, cpad(W0), cpad(T1cat), cpad(T2cat), cpad(Wg1rep), cpad(Wg2rep),
         cpad(P1), cpad(P2), cpad(Wr2), brwd], axis=0)

    full = lambda shape: pl.BlockSpec(shape, lambda g: (0,) * len(shape))
    out = pl.pallas_call(
        _block_kernel,
        grid=(G // B,),
        in_specs=[
            pl.BlockSpec((B * NPG, D), lambda g: (g, 0)),
            pl.BlockSpec((B * NPG, 3), lambda g: (g, 0)),
            pl.BlockSpec((3, B * PAD), lambda g: (0, g)),
            full((752, D)),
        ],
        out_specs=pl.BlockSpec((B * NPG, D), lambda g: (g, 0)),
        out_shape=jax.ShapeDtypeStruct((N, D), jnp.float32),
        compiler_params=pltpu.CompilerParams(
            dimension_semantics=("parallel",)),
    )(nodes, pos, posT, Wpack)

    return out
